# Initial kernel scaffold; baseline (speedup 1.0000x reference)
import jax
import jax.numpy as jnp
from jax.experimental import pallas as pl


def kernel(x, table):
    raise NotImplementedError("write your pallas kernel here")


# speedup vs baseline: 1.8391x; 1.8391x over previous
import functools
import numpy as np
import jax
import jax.numpy as jnp
from jax import lax
from jax.experimental import pallas as pl
from jax.experimental.pallas import tpu as pltpu
from jax.experimental.pallas import tpu_sc as plsc

_B, _L, _D = 16384, 200, 32
_N = _B * _L
_NW = 32
_PER_W = _N // _NW
_C = 2048
_CHUNKS = _PER_W // _C
_GN = 128
_LANES = 16

_BINS = np.linspace(-3.15, 3.15, 64).astype(np.float32)
_FMAX = np.finfo(np.float32).max
_EBHI = np.concatenate([_BINS, [_FMAX] * 64]).astype(np.float32)
_EBLO = np.concatenate([[-_FMAX], _BINS, [_FMAX] * 63]).astype(np.float32)
_LO = np.float32(_BINS[0])
_INV = np.float32(1.0 / ((3.15 - (-3.15)) / 63))


def _sc_body(x_hbm, table_hbm, eblo_hbm, ebhi_hbm, out_hbm,
             x_v, idx_v, rows_v, eblo_v, ebhi_v, sem):
    wid = lax.axis_index("s") * 2 + lax.axis_index("c")
    base = wid * _PER_W

    pltpu.sync_copy(eblo_hbm, eblo_v)
    pltpu.sync_copy(ebhi_hbm, ebhi_v)

    def chunk_body(ci, carry):
        off = base + ci * _C
        pltpu.sync_copy(x_hbm.at[pl.ds(off, _C)], x_v)

        def vec_body(i, c2):
            xv = x_v[pl.ds(i * _LANES, _LANES)]
            p = (xv - _LO) * _INV
            p = jnp.clip(p, -100.0, 100.0)
            te = jnp.clip(p.astype(jnp.int32) + 1, 0, 64)
            bhi = plsc.load_gather(ebhi_v, [te])
            blo = plsc.load_gather(eblo_v, [te])
            t = te + (xv >= bhi).astype(jnp.int32) - (xv < blo).astype(jnp.int32)
            idx = jnp.clip(t, 1, 64)
            idx = jnp.where(xv != xv, 0, idx)
            idx_v[pl.ds(i * _LANES, _LANES)] = idx
            return c2

        lax.fori_loop(0, _C // _LANES, vec_body, 0)

        handles = []
        for g in range(_C // _GN):
            h = pltpu.async_copy(
                table_hbm.at[idx_v.at[pl.ds(g * _GN, _GN)]],
                rows_v.at[pl.ds(g * _GN, _GN)],
                sem,
            )
            handles.append(h)
        for h in handles:
            h.wait()

        pltpu.sync_copy(rows_v, out_hbm.at[pl.ds(off, _C)])
        return carry

    lax.fori_loop(0, _CHUNKS, chunk_body, 0)


@jax.jit
def kernel(x, table):
    mesh = plsc.VectorSubcoreMesh(core_axis_name="c", subcore_axis_name="s")
    call = pl.kernel(
        _sc_body,
        out_type=jax.ShapeDtypeStruct((_N, _D), jnp.float32),
        mesh=mesh,
        compiler_params=pltpu.CompilerParams(
            needs_layout_passes=False, use_tc_tiling_on_sc=False),
        scratch_types=[
            pltpu.VMEM((_C,), jnp.float32),
            pltpu.VMEM((_C,), jnp.int32),
            pltpu.VMEM((_C, _D), jnp.float32),
            pltpu.VMEM((128,), jnp.float32),
            pltpu.VMEM((128,), jnp.float32),
            pltpu.SemaphoreType.DMA,
        ],
    )
    out = call(x.reshape(_N), table, jnp.asarray(_EBLO), jnp.asarray(_EBHI))
    return out.reshape(_B, _L, _D)
